# baseline (speedup 1.0000x reference)
import functools

import jax
import jax.numpy as jnp
from jax import lax
from jax.experimental import pallas as pl
from jax.experimental.pallas import tpu as pltpu
from jax.experimental.pallas import tpu_sc as plsc

NC = 2
NS = 16
LANES = 16
NW = NC * NS


def _make_sc_kernel(N, D, T, chunk, n_chunks):
    rows_per_w = N // NW
    mesh = plsc.VectorSubcoreMesh(core_axis_name="c", subcore_axis_name="s")

    @functools.partial(
        pl.kernel,
        out_type=jax.ShapeDtypeStruct((N, D), jnp.float32),
        mesh=mesh,
        scratch_types=[
            pltpu.VMEM((n_chunks, chunk), jnp.int32),
            pltpu.VMEM((chunk, D), jnp.float32),
            pltpu.VMEM((chunk, D), jnp.float32),
            pltpu.SemaphoreType.DMA,
            pltpu.SemaphoreType.DMA,
        ],
    )
    def sc_kernel(tok_hbm, idx_hbm, pos_hbm, out_hbm, idx_v, tok_buf,
                  pos_buf, gsem, psem):
        wid = lax.axis_index("s") * NC + lax.axis_index("c")
        base = wid * rows_per_w
        pos_base = lax.rem(base, T)
        pltpu.sync_copy(idx_hbm.at[wid], idx_v)
        for c in range(n_chunks):
            g = pltpu.async_copy(tok_hbm.at[idx_v.at[c]], tok_buf, gsem)
            p = pltpu.async_copy(
                pos_hbm.at[pl.ds(pos_base + c * chunk, chunk)], pos_buf, psem)
            g.wait()
            p.wait()

            @pl.loop(0, chunk)
            def _(r):
                for j in range(D // LANES):
                    sl = pl.ds(j * LANES, LANES)
                    tok_buf[r, sl] = tok_buf[r, sl] + pos_buf[r, sl]

            pltpu.sync_copy(tok_buf, out_hbm.at[pl.ds(base + c * chunk, chunk)])

    return sc_kernel


def kernel(idx, tok_table, pos_table):
    B, T = idx.shape
    V, D = tok_table.shape
    N = B * T
    chunk = 32
    n_chunks = N // (NW * chunk)
    idx3 = idx.astype(jnp.int32).reshape(NW, n_chunks, chunk)
    f = _make_sc_kernel(N, D, T, chunk, n_chunks)
    out = f(tok_table, idx3, pos_table)
    return out.reshape(B, T, D)
